# Initial kernel scaffold; baseline (speedup 1.0000x reference)
import jax
import jax.numpy as jnp
from jax.experimental import pallas as pl


def kernel(x, edge_index, W1, b1, mean, log_std_dev, W2, b2, epsilon):
    raise NotImplementedError("write your pallas kernel here")


# speedup vs baseline: 17.0659x; 17.0659x over previous
import functools

import jax
import jax.numpy as jnp
from jax import lax
from jax.experimental import pallas as pl
from jax.experimental.pallas import tpu as pltpu
from jax.experimental.pallas import tpu_sc as plsc

_CH = 128


def _sc_mesh():
    return plsc.VectorSubcoreMesh(core_axis_name="c", subcore_axis_name="s")


def _deg_partials(dst, ones_rows, zeros16, *, n, e):
    info = plsc.get_sparse_core_info()
    nc, ns = info.num_cores, info.num_subcores
    nchunks = e // _CH
    per_core = nchunks // nc
    kmax = (per_core + ns - 1) // ns
    nio = n // 1000

    def body(dst_hbm, ones_hbm, zeros_hbm, out_hbm, dstbuf, onesbuf, acc):
        c = lax.axis_index("c")
        s = lax.axis_index("s")
        rs = s * 1000
        pltpu.sync_copy(ones_hbm, onesbuf)

        @pl.when(s < nio)
        def _():
            pltpu.sync_copy(zeros_hbm.at[pl.ds(rs, 1000)],
                            acc.at[pl.ds(rs, 1000)])

        plsc.subcore_barrier()

        def step(k, carry):
            rel = k * ns + s

            @pl.when(rel < per_core)
            def _():
                off = (c * per_core + rel) * _CH
                pltpu.sync_copy(dst_hbm.at[pl.ds(off, _CH)], dstbuf)
                pltpu.sync_copy(onesbuf, acc.at[dstbuf], add=True)

            return carry

        lax.fori_loop(0, kmax, step, 0)
        plsc.subcore_barrier()

        @pl.when(s < nio)
        def _():
            pltpu.sync_copy(acc.at[pl.ds(rs, 1000)],
                            out_hbm.at[c, pl.ds(rs, 1000)])

    f = pl.kernel(
        body,
        out_type=jax.ShapeDtypeStruct((nc, n, 16), jnp.float32),
        mesh=_sc_mesh(),
        scratch_types=[
            pltpu.VMEM((_CH,), jnp.int32),
            pltpu.VMEM((_CH, 16), jnp.float32),
            pltpu.VMEM_SHARED((n, 16), jnp.float32),
        ],
        compiler_params=pltpu.CompilerParams(use_tc_tiling_on_sc=False),
    )
    return f(dst, ones_rows, zeros16)


def _edge_agg(vals, src, dst, *, n, d, e):
    info = plsc.get_sparse_core_info()
    nc, ns = info.num_cores, info.num_subcores
    nchunks = e // _CH
    per_core = nchunks // nc
    kmax = (per_core + ns - 1) // ns
    nio = n // 1000

    def body(vals_hbm, src_hbm, dst_hbm, out_hbm, srcbuf, dstbuf, rows, acc,
             gsem):
        c = lax.axis_index("c")
        s = lax.axis_index("s")
        rs = s * 1000

        @pl.when(s < nio)
        def _():
            pltpu.sync_copy(vals_hbm.at[pl.ds(rs, 1000)],
                            acc.at[pl.ds(rs, 1000)])

        plsc.subcore_barrier()

        def step(k, carry):
            rel = k * ns + s

            @pl.when(rel < per_core)
            def _():
                off = (c * per_core + rel) * _CH
                pltpu.sync_copy(src_hbm.at[pl.ds(off, _CH)], srcbuf)
                pltpu.sync_copy(dst_hbm.at[pl.ds(off, _CH)], dstbuf)
                pltpu.async_copy(vals_hbm.at[srcbuf], rows, gsem).wait()
                pltpu.sync_copy(rows, acc.at[dstbuf], add=True)

            return carry

        lax.fori_loop(0, kmax, step, 0)
        plsc.subcore_barrier()

        @pl.when(s < nio)
        def _():
            pltpu.sync_copy(acc.at[pl.ds(rs, 1000)],
                            out_hbm.at[c, pl.ds(rs, 1000)])

    f = pl.kernel(
        body,
        out_type=jax.ShapeDtypeStruct((nc, n, d), jnp.float32),
        mesh=_sc_mesh(),
        scratch_types=[
            pltpu.VMEM((_CH,), jnp.int32),
            pltpu.VMEM((_CH,), jnp.int32),
            pltpu.VMEM((_CH, d), jnp.float32),
            pltpu.VMEM_SHARED((n, d), jnp.float32),
            pltpu.SemaphoreType.DMA,
        ],
        compiler_params=pltpu.CompilerParams(use_tc_tiling_on_sc=False),
    )
    return f(vals, src, dst)


_BM = 1000


def _dinv_from_deg(deg_ref):
    deg = jnp.sum(deg_ref[...], axis=(0, 2)) + 1.0
    return lax.rsqrt(deg)[:, None]


def _l1(x, w1, degp, *, n, din, hid):
    def body(x_ref, w_ref, deg_ref, hs_ref):
        dinv = _dinv_from_deg(deg_ref)
        h = jnp.dot(x_ref[...], w_ref[...], preferred_element_type=jnp.float32)
        hs_ref[...] = h * dinv

    return pl.pallas_call(
        body,
        grid=(n // _BM,),
        in_specs=[
            pl.BlockSpec((_BM, din), lambda i: (i, 0)),
            pl.BlockSpec((din, hid), lambda i: (0, 0)),
            pl.BlockSpec((2, _BM, 16), lambda i: (0, i, 0)),
        ],
        out_specs=pl.BlockSpec((_BM, hid), lambda i: (i, 0)),
        out_shape=jax.ShapeDtypeStruct((n, hid), jnp.float32),
    )(x, w1, degp)


def _mid(p, hs, eps, degp, w2a, w2b, b1, mean, lsd, *, n, hid, dout):
    def body(p_ref, hs_ref, eps_ref, deg_ref, w2a_ref, w2b_ref, b1_ref,
             mean_ref, lsd_ref, out_ref):
        dinv = _dinv_from_deg(deg_ref)
        agg = p_ref[0] + p_ref[1] - hs_ref[...]
        h = jnp.maximum(dinv * agg + b1_ref[...], 0.0)
        c = jnp.exp(lsd_ref[...]) * eps_ref[...] + mean_ref[...]
        g = (jnp.dot(h - c, w2a_ref[...], preferred_element_type=jnp.float32)
             + jnp.dot(c, w2b_ref[...], preferred_element_type=jnp.float32))
        out_ref[...] = g * dinv

    return pl.pallas_call(
        body,
        grid=(n // _BM,),
        in_specs=[
            pl.BlockSpec((2, _BM, hid), lambda i: (0, i, 0)),
            pl.BlockSpec((_BM, hid), lambda i: (i, 0)),
            pl.BlockSpec((_BM, hid), lambda i: (i, 0)),
            pl.BlockSpec((2, _BM, 16), lambda i: (0, i, 0)),
            pl.BlockSpec((hid, dout), lambda i: (0, 0)),
            pl.BlockSpec((hid, dout), lambda i: (0, 0)),
            pl.BlockSpec((1, hid), lambda i: (0, 0)),
            pl.BlockSpec((1, hid), lambda i: (0, 0)),
            pl.BlockSpec((1, hid), lambda i: (0, 0)),
        ],
        out_specs=pl.BlockSpec((_BM, dout), lambda i: (i, 0)),
        out_shape=jax.ShapeDtypeStruct((n, dout), jnp.float32),
    )(p, hs, eps, degp, w2a, w2b, b1, mean, lsd)


def _fin(q, gs, degp, b2, *, n, dout):
    def body(q_ref, gs_ref, deg_ref, b2_ref, out_ref):
        dinv = _dinv_from_deg(deg_ref)
        agg = q_ref[0] + q_ref[1] - gs_ref[...]
        out_ref[...] = dinv * agg + b2_ref[...]

    return pl.pallas_call(
        body,
        grid=(n // _BM,),
        in_specs=[
            pl.BlockSpec((2, _BM, dout), lambda i: (0, i, 0)),
            pl.BlockSpec((_BM, dout), lambda i: (i, 0)),
            pl.BlockSpec((2, _BM, 16), lambda i: (0, i, 0)),
            pl.BlockSpec((1, dout), lambda i: (0, 0)),
        ],
        out_specs=pl.BlockSpec((_BM, dout), lambda i: (i, 0)),
        out_shape=jax.ShapeDtypeStruct((n, dout), jnp.float32),
    )(q, gs, degp, b2)


def kernel(x, edge_index, W1, b1, mean, log_std_dev, W2, b2, epsilon):
    n, din = x.shape
    hid = W1.shape[1]
    dout = W2.shape[1]
    e = edge_index.shape[1]

    src = edge_index[0]
    dst = edge_index[1]
    ones_rows = jnp.full((_CH, 16), 1.0 / 16.0, dtype=jnp.float32)
    zeros16 = jnp.zeros((n, 16), dtype=jnp.float32)

    degp = _deg_partials(dst, ones_rows, zeros16, n=n, e=e)
    hs = _l1(x, W1, degp, n=n, din=din, hid=hid)
    p = _edge_agg(hs, src, dst, n=n, d=hid, e=e)
    gs = _mid(p, hs, epsilon, degp, W2[:hid], W2[hid:],
              b1.reshape(1, hid), mean.reshape(1, hid),
              log_std_dev.reshape(1, hid), n=n, hid=hid, dout=dout)
    q = _edge_agg(gs, src, dst, n=n, d=dout, e=e)
    return _fin(q, gs, degp, b2.reshape(1, dout), n=n, dout=dout)


# speedup vs baseline: 33.8288x; 1.9822x over previous
import functools

import jax
import jax.numpy as jnp
from jax import lax
from jax.experimental import pallas as pl
from jax.experimental.pallas import tpu as pltpu
from jax.experimental.pallas import tpu_sc as plsc

_CH = 128


def _sc_mesh():
    return plsc.VectorSubcoreMesh(core_axis_name="c", subcore_axis_name="s")


def _deg_partials(dst2d, ones_rows, zeros16, *, n, e):
    info = plsc.get_sparse_core_info()
    nc, ns = info.num_cores, info.num_subcores
    nchunks = e // _CH
    per_core = nchunks // nc
    per_sub = per_core // ns
    nextra = per_core - per_sub * ns
    rows_io = n // ns
    pipe = 4

    def body(dst_hbm, ones_hbm, zeros_hbm, out_hbm, dstidx, exdst, onesbuf,
             acc, ssem):
        c = lax.axis_index("c")
        s = lax.axis_index("s")
        rs = s * rows_io
        cb = c * per_core + s * per_sub
        pltpu.sync_copy(ones_hbm, onesbuf)
        pltpu.sync_copy(dst_hbm.at[pl.ds(cb, per_sub)], dstidx)
        pltpu.sync_copy(zeros_hbm.at[pl.ds(rs, rows_io)],
                        acc.at[pl.ds(rs, rows_io)])
        plsc.subcore_barrier()

        def step(k, carry):
            pltpu.async_copy(onesbuf, acc.at[dstidx.at[k]], ssem, add=True)

            @pl.when(k >= pipe)
            def _():
                pltpu.make_async_copy(
                    onesbuf, acc.at[dstidx.at[k]], ssem).wait()

            return carry

        lax.fori_loop(0, per_sub, step, 0)
        for j in range(pipe):
            pltpu.make_async_copy(onesbuf, acc.at[dstidx.at[j]], ssem).wait()

        @pl.when(s < nextra)
        def _():
            ex = c * per_core + ns * per_sub + s
            pltpu.sync_copy(dst_hbm.at[ex], exdst)
            pltpu.sync_copy(onesbuf, acc.at[exdst], add=True)

        plsc.subcore_barrier()
        pltpu.sync_copy(acc.at[pl.ds(rs, rows_io)],
                        out_hbm.at[c, pl.ds(rs, rows_io)])

    f = pl.kernel(
        body,
        out_type=jax.ShapeDtypeStruct((nc, n, 16), jnp.float32),
        mesh=_sc_mesh(),
        scratch_types=[
            pltpu.VMEM((per_sub, _CH), jnp.int32),
            pltpu.VMEM((_CH,), jnp.int32),
            pltpu.VMEM((_CH, 16), jnp.float32),
            pltpu.VMEM_SHARED((n, 16), jnp.float32),
            pltpu.SemaphoreType.DMA,
        ],
        compiler_params=pltpu.CompilerParams(use_tc_tiling_on_sc=False),
    )
    return f(dst2d, ones_rows, zeros16)


def _edge_agg(vals, src2d, dst2d, *, n, d, e):
    info = plsc.get_sparse_core_info()
    nc, ns = info.num_cores, info.num_subcores
    nchunks = e // _CH
    per_core = nchunks // nc
    per_sub = per_core // ns
    nextra = per_core - per_sub * ns
    nrounds = per_sub // 3
    tail0 = nrounds * 3
    rows_io = n // ns

    def body(vals_hbm, src_hbm, dst_hbm, out_hbm, sidx, didx, exsrc, exdst,
             rows, acc, isem, gsem, ssem):
        c = lax.axis_index("c")
        s = lax.axis_index("s")
        rs = s * rows_io
        cb = c * per_core + s * per_sub
        pltpu.sync_copy(vals_hbm.at[pl.ds(rs, rows_io)],
                        acc.at[pl.ds(rs, rows_io)])
        plsc.subcore_barrier()

        def fire_idx(k, j):
            pltpu.async_copy(src_hbm.at[cb + k], sidx.at[j], isem)
            pltpu.async_copy(dst_hbm.at[cb + k], didx.at[j], isem)

        def wait_idx(k, j):
            pltpu.make_async_copy(src_hbm.at[cb + k], sidx.at[j], isem).wait()
            pltpu.make_async_copy(dst_hbm.at[cb + k], didx.at[j], isem).wait()

        def fire_g(j):
            pltpu.async_copy(vals_hbm.at[sidx.at[j]], rows.at[j], gsem)

        def wait_g(j):
            pltpu.make_async_copy(vals_hbm.at[sidx.at[j]], rows.at[j],
                                  gsem).wait()

        def fire_s(j):
            pltpu.async_copy(rows.at[j], acc.at[didx.at[j]], ssem, add=True)

        def wait_s(j):
            pltpu.make_async_copy(rows.at[j], acc.at[didx.at[j]], ssem).wait()

        if nrounds > 0:
            fire_idx(0, 0)
            if per_sub > 1:
                fire_idx(1, 1)
            wait_idx(0, 0)
            fire_g(0)

            def round_(g, carry):
                for b in range(3):
                    k = g * 3 + b
                    jp1, jp2 = (b + 1) % 3, (b + 2) % 3

                    @pl.when(k + 1 < per_sub)
                    def _():
                        wait_idx(k + 1, jp1)
                        fire_g(jp1)

                    wait_g(b)
                    fire_s(b)

                    @pl.when(k >= 1)
                    def _():
                        wait_s(jp2)

                    @pl.when(k + 2 < per_sub)
                    def _():
                        fire_idx(k + 2, jp2)

                return carry

            lax.fori_loop(0, nrounds, round_, 0)
            wait_s((tail0 - 1) % 3)

        def tail(k, carry):
            pltpu.sync_copy(src_hbm.at[cb + k], sidx.at[0])
            pltpu.sync_copy(dst_hbm.at[cb + k], didx.at[0])
            pltpu.async_copy(vals_hbm.at[sidx.at[0]], rows.at[0],
                             gsem).wait()
            pltpu.sync_copy(rows.at[0], acc.at[didx.at[0]], add=True)
            return carry

        lax.fori_loop(tail0, per_sub, tail, 0)

        @pl.when(s < nextra)
        def _():
            ex = c * per_core + ns * per_sub + s
            pltpu.sync_copy(src_hbm.at[ex], exsrc)
            pltpu.sync_copy(dst_hbm.at[ex], exdst)
            pltpu.async_copy(vals_hbm.at[exsrc], rows.at[0], gsem).wait()
            pltpu.sync_copy(rows.at[0], acc.at[exdst], add=True)

        plsc.subcore_barrier()
        pltpu.sync_copy(acc.at[pl.ds(rs, rows_io)],
                        out_hbm.at[c, pl.ds(rs, rows_io)])

    f = pl.kernel(
        body,
        out_type=jax.ShapeDtypeStruct((nc, n, d), jnp.float32),
        mesh=_sc_mesh(),
        scratch_types=[
            pltpu.VMEM((3, _CH), jnp.int32),
            pltpu.VMEM((3, _CH), jnp.int32),
            pltpu.VMEM((_CH,), jnp.int32),
            pltpu.VMEM((_CH,), jnp.int32),
            pltpu.VMEM((3, _CH, d), jnp.float32),
            pltpu.VMEM_SHARED((n, d), jnp.float32),
            pltpu.SemaphoreType.DMA,
            pltpu.SemaphoreType.DMA,
            pltpu.SemaphoreType.DMA,
        ],
        compiler_params=pltpu.CompilerParams(use_tc_tiling_on_sc=False),
    )
    return f(vals, src2d, dst2d)


_BM = 1000


def _dinv_from_deg(deg_ref):
    deg = jnp.sum(deg_ref[...], axis=(0, 2)) + 1.0
    return lax.rsqrt(deg)[:, None]


def _l1(x, w1, degp, *, n, din, hid):
    def body(x_ref, w_ref, deg_ref, hs_ref):
        dinv = _dinv_from_deg(deg_ref)
        h = jnp.dot(x_ref[...], w_ref[...], preferred_element_type=jnp.float32)
        hs_ref[...] = h * dinv

    return pl.pallas_call(
        body,
        grid=(n // _BM,),
        in_specs=[
            pl.BlockSpec((_BM, din), lambda i: (i, 0)),
            pl.BlockSpec((din, hid), lambda i: (0, 0)),
            pl.BlockSpec((2, _BM, 16), lambda i: (0, i, 0)),
        ],
        out_specs=pl.BlockSpec((_BM, hid), lambda i: (i, 0)),
        out_shape=jax.ShapeDtypeStruct((n, hid), jnp.float32),
    )(x, w1, degp)


def _mid(p, hs, eps, degp, w2a, w2b, b1, mean, lsd, *, n, hid, dout):
    def body(p_ref, hs_ref, eps_ref, deg_ref, w2a_ref, w2b_ref, b1_ref,
             mean_ref, lsd_ref, out_ref):
        dinv = _dinv_from_deg(deg_ref)
        agg = p_ref[0] + p_ref[1] - hs_ref[...]
        h = jnp.maximum(dinv * agg + b1_ref[...], 0.0)
        c = jnp.exp(lsd_ref[...]) * eps_ref[...] + mean_ref[...]
        g = (jnp.dot(h - c, w2a_ref[...], preferred_element_type=jnp.float32)
             + jnp.dot(c, w2b_ref[...], preferred_element_type=jnp.float32))
        out_ref[...] = g * dinv

    return pl.pallas_call(
        body,
        grid=(n // _BM,),
        in_specs=[
            pl.BlockSpec((2, _BM, hid), lambda i: (0, i, 0)),
            pl.BlockSpec((_BM, hid), lambda i: (i, 0)),
            pl.BlockSpec((_BM, hid), lambda i: (i, 0)),
            pl.BlockSpec((2, _BM, 16), lambda i: (0, i, 0)),
            pl.BlockSpec((hid, dout), lambda i: (0, 0)),
            pl.BlockSpec((hid, dout), lambda i: (0, 0)),
            pl.BlockSpec((1, hid), lambda i: (0, 0)),
            pl.BlockSpec((1, hid), lambda i: (0, 0)),
            pl.BlockSpec((1, hid), lambda i: (0, 0)),
        ],
        out_specs=pl.BlockSpec((_BM, dout), lambda i: (i, 0)),
        out_shape=jax.ShapeDtypeStruct((n, dout), jnp.float32),
    )(p, hs, eps, degp, w2a, w2b, b1, mean, lsd)


def _fin(q, gs, degp, b2, *, n, dout):
    def body(q_ref, gs_ref, deg_ref, b2_ref, out_ref):
        dinv = _dinv_from_deg(deg_ref)
        agg = q_ref[0] + q_ref[1] - gs_ref[...]
        out_ref[...] = dinv * agg + b2_ref[...]

    return pl.pallas_call(
        body,
        grid=(n // _BM,),
        in_specs=[
            pl.BlockSpec((2, _BM, dout), lambda i: (0, i, 0)),
            pl.BlockSpec((_BM, dout), lambda i: (i, 0)),
            pl.BlockSpec((2, _BM, 16), lambda i: (0, i, 0)),
            pl.BlockSpec((1, dout), lambda i: (0, 0)),
        ],
        out_specs=pl.BlockSpec((_BM, dout), lambda i: (i, 0)),
        out_shape=jax.ShapeDtypeStruct((n, dout), jnp.float32),
    )(q, gs, degp, b2)


def kernel(x, edge_index, W1, b1, mean, log_std_dev, W2, b2, epsilon):
    n, din = x.shape
    hid = W1.shape[1]
    dout = W2.shape[1]
    e = edge_index.shape[1]

    src2d = edge_index[0].reshape(e // _CH, _CH)
    dst2d = edge_index[1].reshape(e // _CH, _CH)
    ones_rows = jnp.full((_CH, 16), 1.0 / 16.0, dtype=jnp.float32)
    zeros16 = jnp.zeros((n, 16), dtype=jnp.float32)

    degp = _deg_partials(dst2d, ones_rows, zeros16, n=n, e=e)
    hs = _l1(x, W1, degp, n=n, din=din, hid=hid)
    p = _edge_agg(hs, src2d, dst2d, n=n, d=hid, e=e)
    gs = _mid(p, hs, epsilon, degp, W2[:hid], W2[hid:],
              b1.reshape(1, hid), mean.reshape(1, hid),
              log_std_dev.reshape(1, hid), n=n, hid=hid, dout=dout)
    q = _edge_agg(gs, src2d, dst2d, n=n, d=dout, e=e)
    return _fin(q, gs, degp, b2.reshape(1, dout), n=n, dout=dout)


# speedup vs baseline: 34.0466x; 1.0064x over previous
import functools

import jax
import jax.numpy as jnp
from jax import lax
from jax.experimental import pallas as pl
from jax.experimental.pallas import tpu as pltpu
from jax.experimental.pallas import tpu_sc as plsc

_CH = 128


def _sc_mesh():
    return plsc.VectorSubcoreMesh(core_axis_name="c", subcore_axis_name="s")


def _deg_partials(dst2d, ones_rows, zeros16, *, n, e):
    info = plsc.get_sparse_core_info()
    nc, ns = info.num_cores, info.num_subcores
    nchunks = e // _CH
    per_core = nchunks // nc
    per_sub = per_core // ns
    nextra = per_core - per_sub * ns
    rows_io = n // ns
    pipe = 4

    def body(dst_hbm, ones_hbm, zeros_hbm, out_hbm, dstidx, exdst, onesbuf,
             acc, ssem):
        c = lax.axis_index("c")
        s = lax.axis_index("s")
        rs = s * rows_io
        cb = c * per_core + s * per_sub
        pltpu.sync_copy(ones_hbm, onesbuf)
        pltpu.sync_copy(dst_hbm.at[pl.ds(cb, per_sub)], dstidx)
        pltpu.sync_copy(zeros_hbm.at[pl.ds(rs, rows_io)],
                        acc.at[pl.ds(rs, rows_io)])
        plsc.subcore_barrier()

        def step(k, carry):
            pltpu.async_copy(onesbuf, acc.at[dstidx.at[k]], ssem, add=True)

            @pl.when(k >= pipe)
            def _():
                pltpu.make_async_copy(
                    onesbuf, acc.at[dstidx.at[k]], ssem).wait()

            return carry

        lax.fori_loop(0, per_sub, step, 0)
        for j in range(pipe):
            pltpu.make_async_copy(onesbuf, acc.at[dstidx.at[j]], ssem).wait()

        @pl.when(s < nextra)
        def _():
            ex = c * per_core + ns * per_sub + s
            pltpu.sync_copy(dst_hbm.at[ex], exdst)
            pltpu.sync_copy(onesbuf, acc.at[exdst], add=True)

        plsc.subcore_barrier()
        pltpu.sync_copy(acc.at[pl.ds(rs, rows_io)],
                        out_hbm.at[c, pl.ds(rs, rows_io)])

    f = pl.kernel(
        body,
        out_type=jax.ShapeDtypeStruct((nc, n, 16), jnp.float32),
        mesh=_sc_mesh(),
        scratch_types=[
            pltpu.VMEM((per_sub, _CH), jnp.int32),
            pltpu.VMEM((_CH,), jnp.int32),
            pltpu.VMEM((_CH, 16), jnp.float32),
            pltpu.VMEM_SHARED((n, 16), jnp.float32),
            pltpu.SemaphoreType.DMA,
        ],
        compiler_params=pltpu.CompilerParams(use_tc_tiling_on_sc=False),
    )
    return f(dst2d, ones_rows, zeros16)


def _edge_agg(vals, eidx, *, n, d, e):
    info = plsc.get_sparse_core_info()
    nc, ns = info.num_cores, info.num_subcores
    nchunks = e // _CH
    per_core = nchunks // nc
    per_sub = per_core // ns
    nextra = per_core - per_sub * ns
    nrounds = per_sub // 3
    tail0 = nrounds * 3
    rows_io = n // ns

    def body(vals_hbm, eidx_hbm, out_hbm, eidx, exidx, rows, acc, isem, gsem,
             ssem):
        c = lax.axis_index("c")
        s = lax.axis_index("s")
        rs = s * rows_io
        cb = c * per_core + s * per_sub
        pltpu.sync_copy(vals_hbm.at[pl.ds(rs, rows_io)],
                        acc.at[pl.ds(rs, rows_io)])
        plsc.subcore_barrier()

        def fire_idx(k, j):
            pltpu.async_copy(eidx_hbm.at[cb + k], eidx.at[j], isem)

        def wait_idx(k, j):
            pltpu.make_async_copy(eidx_hbm.at[cb + k], eidx.at[j],
                                  isem).wait()

        def fire_g(j):
            pltpu.async_copy(vals_hbm.at[eidx.at[j, 0]], rows.at[j], gsem)

        def wait_g(j):
            pltpu.make_async_copy(vals_hbm.at[eidx.at[j, 0]], rows.at[j],
                                  gsem).wait()

        def fire_s(j):
            pltpu.async_copy(rows.at[j], acc.at[eidx.at[j, 1]], ssem,
                             add=True)

        def wait_s(j):
            pltpu.make_async_copy(rows.at[j], acc.at[eidx.at[j, 1]],
                                  ssem).wait()

        if nrounds > 0:
            fire_idx(0, 0)
            if per_sub > 1:
                fire_idx(1, 1)
            wait_idx(0, 0)
            fire_g(0)

            def round_(g, carry):
                for b in range(3):
                    k = g * 3 + b
                    jp1, jp2 = (b + 1) % 3, (b + 2) % 3

                    @pl.when(k + 1 < per_sub)
                    def _():
                        wait_idx(k + 1, jp1)
                        fire_g(jp1)

                    wait_g(b)
                    fire_s(b)

                    @pl.when(k >= 1)
                    def _():
                        wait_s(jp2)

                    @pl.when(k + 2 < per_sub)
                    def _():
                        fire_idx(k + 2, jp2)

                return carry

            lax.fori_loop(0, nrounds, round_, 0)
            wait_s((tail0 - 1) % 3)

        def tail(k, carry):
            pltpu.sync_copy(eidx_hbm.at[cb + k], eidx.at[0])
            pltpu.async_copy(vals_hbm.at[eidx.at[0, 0]], rows.at[0],
                             gsem).wait()
            pltpu.sync_copy(rows.at[0], acc.at[eidx.at[0, 1]], add=True)
            return carry

        lax.fori_loop(tail0, per_sub, tail, 0)

        @pl.when(s < nextra)
        def _():
            ex = c * per_core + ns * per_sub + s
            pltpu.sync_copy(eidx_hbm.at[ex], exidx)
            pltpu.async_copy(vals_hbm.at[exidx.at[0]], rows.at[0],
                             gsem).wait()
            pltpu.sync_copy(rows.at[0], acc.at[exidx.at[1]], add=True)

        plsc.subcore_barrier()
        pltpu.sync_copy(acc.at[pl.ds(rs, rows_io)],
                        out_hbm.at[c, pl.ds(rs, rows_io)])

    f = pl.kernel(
        body,
        out_type=jax.ShapeDtypeStruct((nc, n, d), jnp.float32),
        mesh=_sc_mesh(),
        scratch_types=[
            pltpu.VMEM((3, 2, _CH), jnp.int32),
            pltpu.VMEM((2, _CH), jnp.int32),
            pltpu.VMEM((3, _CH, d), jnp.float32),
            pltpu.VMEM_SHARED((n, d), jnp.float32),
            pltpu.SemaphoreType.DMA,
            pltpu.SemaphoreType.DMA,
            pltpu.SemaphoreType.DMA,
        ],
        compiler_params=pltpu.CompilerParams(use_tc_tiling_on_sc=False),
    )
    return f(vals, eidx)


_BM = 1000


def _dinv_from_deg(deg_ref):
    deg = jnp.sum(deg_ref[...], axis=(0, 2)) + 1.0
    return lax.rsqrt(deg)[:, None]


def _l1(x, w1, degp, *, n, din, hid):
    def body(x_ref, w_ref, deg_ref, hs_ref):
        dinv = _dinv_from_deg(deg_ref)
        h = jnp.dot(x_ref[...], w_ref[...], preferred_element_type=jnp.float32)
        hs_ref[...] = h * dinv

    return pl.pallas_call(
        body,
        grid=(n // _BM,),
        in_specs=[
            pl.BlockSpec((_BM, din), lambda i: (i, 0)),
            pl.BlockSpec((din, hid), lambda i: (0, 0)),
            pl.BlockSpec((2, _BM, 16), lambda i: (0, i, 0)),
        ],
        out_specs=pl.BlockSpec((_BM, hid), lambda i: (i, 0)),
        out_shape=jax.ShapeDtypeStruct((n, hid), jnp.float32),
    )(x, w1, degp)


def _mid(p, hs, eps, degp, w2a, w2b, b1, mean, lsd, *, n, hid, dout):
    def body(p_ref, hs_ref, eps_ref, deg_ref, w2a_ref, w2b_ref, b1_ref,
             mean_ref, lsd_ref, out_ref):
        dinv = _dinv_from_deg(deg_ref)
        agg = p_ref[0] + p_ref[1] - hs_ref[...]
        h = jnp.maximum(dinv * agg + b1_ref[...], 0.0)
        c = jnp.exp(lsd_ref[...]) * eps_ref[...] + mean_ref[...]
        g = (jnp.dot(h - c, w2a_ref[...], preferred_element_type=jnp.float32)
             + jnp.dot(c, w2b_ref[...], preferred_element_type=jnp.float32))
        out_ref[...] = g * dinv

    return pl.pallas_call(
        body,
        grid=(n // _BM,),
        in_specs=[
            pl.BlockSpec((2, _BM, hid), lambda i: (0, i, 0)),
            pl.BlockSpec((_BM, hid), lambda i: (i, 0)),
            pl.BlockSpec((_BM, hid), lambda i: (i, 0)),
            pl.BlockSpec((2, _BM, 16), lambda i: (0, i, 0)),
            pl.BlockSpec((hid, dout), lambda i: (0, 0)),
            pl.BlockSpec((hid, dout), lambda i: (0, 0)),
            pl.BlockSpec((1, hid), lambda i: (0, 0)),
            pl.BlockSpec((1, hid), lambda i: (0, 0)),
            pl.BlockSpec((1, hid), lambda i: (0, 0)),
        ],
        out_specs=pl.BlockSpec((_BM, dout), lambda i: (i, 0)),
        out_shape=jax.ShapeDtypeStruct((n, dout), jnp.float32),
    )(p, hs, eps, degp, w2a, w2b, b1, mean, lsd)


def _fin(q, gs, degp, b2, *, n, dout):
    def body(q_ref, gs_ref, deg_ref, b2_ref, out_ref):
        dinv = _dinv_from_deg(deg_ref)
        agg = q_ref[0] + q_ref[1] - gs_ref[...]
        out_ref[...] = dinv * agg + b2_ref[...]

    return pl.pallas_call(
        body,
        grid=(n // _BM,),
        in_specs=[
            pl.BlockSpec((2, _BM, dout), lambda i: (0, i, 0)),
            pl.BlockSpec((_BM, dout), lambda i: (i, 0)),
            pl.BlockSpec((2, _BM, 16), lambda i: (0, i, 0)),
            pl.BlockSpec((1, dout), lambda i: (0, 0)),
        ],
        out_specs=pl.BlockSpec((_BM, dout), lambda i: (i, 0)),
        out_shape=jax.ShapeDtypeStruct((n, dout), jnp.float32),
    )(q, gs, degp, b2)


def kernel(x, edge_index, W1, b1, mean, log_std_dev, W2, b2, epsilon):
    n, din = x.shape
    hid = W1.shape[1]
    dout = W2.shape[1]
    e = edge_index.shape[1]

    dst2d = edge_index[1].reshape(e // _CH, _CH)
    eidx = jnp.swapaxes(edge_index.reshape(2, e // _CH, _CH), 0, 1)
    ones_rows = jnp.full((_CH, 16), 1.0 / 16.0, dtype=jnp.float32)
    zeros16 = jnp.zeros((n, 16), dtype=jnp.float32)

    degp = _deg_partials(dst2d, ones_rows, zeros16, n=n, e=e)
    hs = _l1(x, W1, degp, n=n, din=din, hid=hid)
    p = _edge_agg(hs, eidx, n=n, d=hid, e=e)
    gs = _mid(p, hs, epsilon, degp, W2[:hid], W2[hid:],
              b1.reshape(1, hid), mean.reshape(1, hid),
              log_std_dev.reshape(1, hid), n=n, hid=hid, dout=dout)
    q = _edge_agg(gs, eidx, n=n, d=dout, e=e)
    return _fin(q, gs, degp, b2.reshape(1, dout), n=n, dout=dout)


# speedup vs baseline: 39.4300x; 1.1581x over previous
import functools

import jax
import jax.numpy as jnp
from jax import lax
from jax.experimental import pallas as pl
from jax.experimental.pallas import tpu as pltpu
from jax.experimental.pallas import tpu_sc as plsc

_CH = 128


def _sc_mesh():
    return plsc.VectorSubcoreMesh(core_axis_name="c", subcore_axis_name="s")


def _deg_partials(eidx, ones_rows, zeros16, *, n, e):
    info = plsc.get_sparse_core_info()
    nc, ns = info.num_cores, info.num_subcores
    nchunks = e // _CH
    per_core = nchunks // nc
    per_sub = per_core // ns
    nextra = per_core - per_sub * ns
    rows_io = n // ns
    pipe = 4

    def body(eidx_hbm, ones_hbm, zeros_hbm, out_hbm, dstidx, exdst, onesbuf,
             acc, ssem):
        c = lax.axis_index("c")
        s = lax.axis_index("s")
        rs = s * rows_io
        cb = c * per_core + s * per_sub
        pltpu.sync_copy(ones_hbm, onesbuf)
        pltpu.sync_copy(eidx_hbm.at[pl.ds(cb, per_sub), 1], dstidx)
        pltpu.sync_copy(zeros_hbm.at[pl.ds(rs, rows_io)],
                        acc.at[pl.ds(rs, rows_io)])
        plsc.subcore_barrier()

        def step(k, carry):
            pltpu.async_copy(onesbuf, acc.at[dstidx.at[k]], ssem, add=True)

            @pl.when(k >= pipe)
            def _():
                pltpu.make_async_copy(
                    onesbuf, acc.at[dstidx.at[k]], ssem).wait()

            return carry

        lax.fori_loop(0, per_sub, step, 0)
        for j in range(pipe):
            pltpu.make_async_copy(onesbuf, acc.at[dstidx.at[j]], ssem).wait()

        @pl.when(s < nextra)
        def _():
            ex = c * per_core + ns * per_sub + s
            pltpu.sync_copy(eidx_hbm.at[ex, 1], exdst)
            pltpu.sync_copy(onesbuf, acc.at[exdst], add=True)

        plsc.subcore_barrier()
        pltpu.sync_copy(acc.at[pl.ds(rs, rows_io)],
                        out_hbm.at[pl.ds(rs, rows_io), pl.ds(c * 16, 16)])

    f = pl.kernel(
        body,
        out_type=jax.ShapeDtypeStruct((n, nc * 16), jnp.float32),
        mesh=_sc_mesh(),
        scratch_types=[
            pltpu.VMEM((per_sub, _CH), jnp.int32),
            pltpu.VMEM((_CH,), jnp.int32),
            pltpu.VMEM((_CH, 16), jnp.float32),
            pltpu.VMEM_SHARED((n, 16), jnp.float32),
            pltpu.SemaphoreType.DMA,
        ],
        compiler_params=pltpu.CompilerParams(use_tc_tiling_on_sc=False),
    )
    return f(eidx, ones_rows, zeros16)


def _edge_agg(vals, eidx, *, n, d, e):
    info = plsc.get_sparse_core_info()
    nc, ns = info.num_cores, info.num_subcores
    nchunks = e // _CH
    per_core = nchunks // nc
    per_sub = per_core // ns
    nextra = per_core - per_sub * ns
    S, G, I = (3, 1, 2) if d > 64 else (6, 2, 3)
    nrounds = per_sub // S
    tail0 = nrounds * S
    rows_io = n // ns
    packed = nc * d <= 128

    def body(vals_hbm, eidx_hbm, out_hbm, eidx, exidx, rows, acc, isem, gsem,
             ssem):
        c = lax.axis_index("c")
        s = lax.axis_index("s")
        rs = s * rows_io
        cb = c * per_core + s * per_sub
        pltpu.sync_copy(vals_hbm.at[pl.ds(rs, rows_io)],
                        acc.at[pl.ds(rs, rows_io)])
        plsc.subcore_barrier()

        def fire_idx(k, j):
            pltpu.async_copy(eidx_hbm.at[cb + k], eidx.at[j], isem)

        def wait_idx(k, j):
            pltpu.make_async_copy(eidx_hbm.at[cb + k], eidx.at[j],
                                  isem).wait()

        def fire_g(j):
            pltpu.async_copy(vals_hbm.at[eidx.at[j, 0]], rows.at[j], gsem)

        def wait_g(j):
            pltpu.make_async_copy(vals_hbm.at[eidx.at[j, 0]], rows.at[j],
                                  gsem).wait()

        def fire_s(j):
            pltpu.async_copy(rows.at[j], acc.at[eidx.at[j, 1]], ssem,
                             add=True)

        def wait_s(j):
            pltpu.make_async_copy(rows.at[j], acc.at[eidx.at[j, 1]],
                                  ssem).wait()

        if nrounds > 0:
            for k0 in range(min(I, tail0)):
                fire_idx(k0, k0)
            for k0 in range(min(G, tail0)):
                wait_idx(k0, k0)
                fire_g(k0)

            def round_(g, carry):
                for b in range(S):
                    k = g * S + b

                    @pl.when(k + G < tail0)
                    def _():
                        wait_idx(k + G, (b + G) % S)
                        fire_g((b + G) % S)

                    wait_g(b)
                    fire_s(b)

                    @pl.when(k + I >= S)
                    def _():
                        wait_s((b + I) % S)

                    @pl.when(k + I < tail0)
                    def _():
                        fire_idx(k + I, (b + I) % S)

                return carry

            lax.fori_loop(0, nrounds, round_, 0)
            for t in range(min(S - I, tail0)):
                wait_s((tail0 - 1 - t) % S)

        def tail(k, carry):
            pltpu.sync_copy(eidx_hbm.at[cb + k], eidx.at[0])
            pltpu.async_copy(vals_hbm.at[eidx.at[0, 0]], rows.at[0],
                             gsem).wait()
            pltpu.sync_copy(rows.at[0], acc.at[eidx.at[0, 1]], add=True)
            return carry

        lax.fori_loop(tail0, per_sub, tail, 0)

        @pl.when(s < nextra)
        def _():
            ex = c * per_core + ns * per_sub + s
            pltpu.sync_copy(eidx_hbm.at[ex], exidx)
            pltpu.async_copy(vals_hbm.at[exidx.at[0]], rows.at[0],
                             gsem).wait()
            pltpu.sync_copy(rows.at[0], acc.at[exidx.at[1]], add=True)

        plsc.subcore_barrier()
        if packed:
            pltpu.sync_copy(acc.at[pl.ds(rs, rows_io)],
                            out_hbm.at[pl.ds(rs, rows_io), pl.ds(c * d, d)])
        else:
            pltpu.sync_copy(acc.at[pl.ds(rs, rows_io)],
                            out_hbm.at[c, pl.ds(rs, rows_io)])

    out_shape = ((n, nc * d) if packed else (nc, n, d))
    f = pl.kernel(
        body,
        out_type=jax.ShapeDtypeStruct(out_shape, jnp.float32),
        mesh=_sc_mesh(),
        scratch_types=[
            pltpu.VMEM((S, 2, _CH), jnp.int32),
            pltpu.VMEM((2, _CH), jnp.int32),
            pltpu.VMEM((S, _CH, d), jnp.float32),
            pltpu.VMEM_SHARED((n, d), jnp.float32),
            pltpu.SemaphoreType.DMA,
            pltpu.SemaphoreType.DMA,
            pltpu.SemaphoreType.DMA,
        ],
        compiler_params=pltpu.CompilerParams(use_tc_tiling_on_sc=False),
    )
    return f(vals, eidx)


_BM = 1000


def _dinv_from_deg(deg_ref):
    deg = jnp.sum(deg_ref[...], axis=1) + 1.0
    return lax.rsqrt(deg)[:, None]


def _l1(x, w1, degp, *, n, din, hid):
    def body(x_ref, w_ref, deg_ref, hs_ref):
        dinv = _dinv_from_deg(deg_ref)
        h = jnp.dot(x_ref[...], w_ref[...], preferred_element_type=jnp.float32)
        hs_ref[...] = h * dinv

    return pl.pallas_call(
        body,
        grid=(n // _BM,),
        in_specs=[
            pl.BlockSpec((_BM, din), lambda i: (i, 0)),
            pl.BlockSpec((din, hid), lambda i: (0, 0)),
            pl.BlockSpec((_BM, 32), lambda i: (i, 0)),
        ],
        out_specs=pl.BlockSpec((_BM, hid), lambda i: (i, 0)),
        out_shape=jax.ShapeDtypeStruct((n, hid), jnp.float32),
    )(x, w1, degp)


def _mid(p, hs, eps, degp, w2a, w2b, b1, mean, lsd, *, n, hid, dout):
    def body(p_ref, hs_ref, eps_ref, deg_ref, w2a_ref, w2b_ref, b1_ref,
             mean_ref, lsd_ref, out_ref):
        dinv = _dinv_from_deg(deg_ref)
        agg = p_ref[0] + p_ref[1] - hs_ref[...]
        h = jnp.maximum(dinv * agg + b1_ref[...], 0.0)
        c = jnp.exp(lsd_ref[...]) * eps_ref[...] + mean_ref[...]
        g = (jnp.dot(h - c, w2a_ref[...], preferred_element_type=jnp.float32)
             + jnp.dot(c, w2b_ref[...], preferred_element_type=jnp.float32))
        out_ref[...] = g * dinv

    return pl.pallas_call(
        body,
        grid=(n // _BM,),
        in_specs=[
            pl.BlockSpec((2, _BM, hid), lambda i: (0, i, 0)),
            pl.BlockSpec((_BM, hid), lambda i: (i, 0)),
            pl.BlockSpec((_BM, hid), lambda i: (i, 0)),
            pl.BlockSpec((_BM, 32), lambda i: (i, 0)),
            pl.BlockSpec((hid, dout), lambda i: (0, 0)),
            pl.BlockSpec((hid, dout), lambda i: (0, 0)),
            pl.BlockSpec((1, hid), lambda i: (0, 0)),
            pl.BlockSpec((1, hid), lambda i: (0, 0)),
            pl.BlockSpec((1, hid), lambda i: (0, 0)),
        ],
        out_specs=pl.BlockSpec((_BM, dout), lambda i: (i, 0)),
        out_shape=jax.ShapeDtypeStruct((n, dout), jnp.float32),
    )(p, hs, eps, degp, w2a, w2b, b1, mean, lsd)


def _fin(q, gs, degp, b2, *, n, dout):
    def body(q_ref, gs_ref, deg_ref, b2_ref, out_ref):
        dinv = _dinv_from_deg(deg_ref)
        agg = q_ref[:, :dout] + q_ref[:, dout:] - gs_ref[...]
        out_ref[...] = dinv * agg + b2_ref[...]

    return pl.pallas_call(
        body,
        grid=(n // _BM,),
        in_specs=[
            pl.BlockSpec((_BM, 2 * dout), lambda i: (i, 0)),
            pl.BlockSpec((_BM, dout), lambda i: (i, 0)),
            pl.BlockSpec((_BM, 32), lambda i: (i, 0)),
            pl.BlockSpec((1, dout), lambda i: (0, 0)),
        ],
        out_specs=pl.BlockSpec((_BM, dout), lambda i: (i, 0)),
        out_shape=jax.ShapeDtypeStruct((n, dout), jnp.float32),
    )(q, gs, degp, b2)


def kernel(x, edge_index, W1, b1, mean, log_std_dev, W2, b2, epsilon):
    n, din = x.shape
    hid = W1.shape[1]
    dout = W2.shape[1]
    e = edge_index.shape[1]

    eidx = jnp.swapaxes(edge_index.reshape(2, e // _CH, _CH), 0, 1)
    ones_rows = jnp.full((_CH, 16), 1.0 / 16.0, dtype=jnp.float32)
    zeros16 = jnp.zeros((n, 16), dtype=jnp.float32)

    degp = _deg_partials(eidx, ones_rows, zeros16, n=n, e=e)
    hs = _l1(x, W1, degp, n=n, din=din, hid=hid)
    p = _edge_agg(hs, eidx, n=n, d=hid, e=e)
    gs = _mid(p, hs, epsilon, degp, W2[:hid], W2[hid:],
              b1.reshape(1, hid), mean.reshape(1, hid),
              log_std_dev.reshape(1, hid), n=n, hid=hid, dout=dout)
    q = _edge_agg(gs, eidx, n=n, d=dout, e=e)
    return _fin(q, gs, degp, b2.reshape(1, dout), n=n, dout=dout)


# speedup vs baseline: 40.2160x; 1.0199x over previous
import functools

import jax
import jax.numpy as jnp
from jax import lax
from jax.experimental import pallas as pl
from jax.experimental.pallas import tpu as pltpu
from jax.experimental.pallas import tpu_sc as plsc

_CH = 128


def _sc_mesh():
    return plsc.VectorSubcoreMesh(core_axis_name="c", subcore_axis_name="s")


def _deg_partials(eidx, ones_rows, zeros16, *, n, e):
    info = plsc.get_sparse_core_info()
    nc, ns = info.num_cores, info.num_subcores
    nchunks = e // _CH
    per_core = nchunks // nc
    per_sub = per_core // ns
    nextra = per_core - per_sub * ns
    rows_io = n // ns
    pipe = 4

    def body(eidx_hbm, ones_hbm, zeros_hbm, out_hbm, dstidx, exdst, onesbuf,
             acc, ssem):
        c = lax.axis_index("c")
        s = lax.axis_index("s")
        rs = s * rows_io
        cb = c * per_core + s * per_sub
        pltpu.sync_copy(ones_hbm, onesbuf)
        pltpu.sync_copy(eidx_hbm.at[pl.ds(cb, per_sub), 1], dstidx)
        pltpu.sync_copy(zeros_hbm.at[pl.ds(rs, rows_io)],
                        acc.at[pl.ds(rs, rows_io)])
        plsc.subcore_barrier()

        def step(k, carry):
            pltpu.async_copy(onesbuf, acc.at[dstidx.at[k]], ssem, add=True)

            @pl.when(k >= pipe)
            def _():
                pltpu.make_async_copy(
                    onesbuf, acc.at[dstidx.at[k]], ssem).wait()

            return carry

        lax.fori_loop(0, per_sub, step, 0)
        for j in range(pipe):
            pltpu.make_async_copy(onesbuf, acc.at[dstidx.at[j]], ssem).wait()

        @pl.when(s < nextra)
        def _():
            ex = c * per_core + ns * per_sub + s
            pltpu.sync_copy(eidx_hbm.at[ex, 1], exdst)
            pltpu.sync_copy(onesbuf, acc.at[exdst], add=True)

        plsc.subcore_barrier()
        pltpu.sync_copy(acc.at[pl.ds(rs, rows_io)],
                        out_hbm.at[pl.ds(rs, rows_io), pl.ds(c * 16, 16)])

    f = pl.kernel(
        body,
        out_type=jax.ShapeDtypeStruct((n, nc * 16), jnp.float32),
        mesh=_sc_mesh(),
        scratch_types=[
            pltpu.VMEM((per_sub, _CH), jnp.int32),
            pltpu.VMEM((_CH,), jnp.int32),
            pltpu.VMEM((_CH, 16), jnp.float32),
            pltpu.VMEM_SHARED((n, 16), jnp.float32),
            pltpu.SemaphoreType.DMA,
        ],
        compiler_params=pltpu.CompilerParams(use_tc_tiling_on_sc=False),
    )
    return f(eidx, ones_rows, zeros16)


def _edge_agg(vals, eidx, *, n, d, e):
    info = plsc.get_sparse_core_info()
    nc, ns = info.num_cores, info.num_subcores
    nchunks = e // _CH
    per_core = nchunks // nc
    per_sub = per_core // ns
    nextra = per_core - per_sub * ns
    S, G, I = (3, 1, 2) if d > 64 else (6, 2, 3)
    nrounds = per_sub // S
    tail0 = nrounds * S
    rows_io = n // ns
    packed = nc * d <= 128

    def body(vals_hbm, eidx_hbm, out_hbm, eidx, exidx, rows, acc, isem, gsem,
             ssem):
        c = lax.axis_index("c")
        s = lax.axis_index("s")
        rs = s * rows_io
        cb = c * per_core + s * per_sub
        pltpu.sync_copy(vals_hbm.at[pl.ds(rs, rows_io)],
                        acc.at[pl.ds(rs, rows_io)])
        plsc.subcore_barrier()

        def fire_idx(k, j):
            pltpu.async_copy(eidx_hbm.at[cb + k], eidx.at[j], isem)

        def wait_idx(k, j):
            pltpu.make_async_copy(eidx_hbm.at[cb + k], eidx.at[j],
                                  isem).wait()

        def fire_g(j):
            pltpu.async_copy(vals_hbm.at[eidx.at[j, 0]], rows.at[j], gsem)

        def wait_g(j):
            pltpu.make_async_copy(vals_hbm.at[eidx.at[j, 0]], rows.at[j],
                                  gsem).wait()

        def fire_s(j):
            pltpu.async_copy(rows.at[j], acc.at[eidx.at[j, 1]], ssem,
                             add=True)

        def wait_s(j):
            pltpu.make_async_copy(rows.at[j], acc.at[eidx.at[j, 1]],
                                  ssem).wait()

        if nrounds > 0:
            for k0 in range(min(I, tail0)):
                fire_idx(k0, k0)
            for k0 in range(min(G, tail0)):
                wait_idx(k0, k0)
                fire_g(k0)

            def round_(g, carry):
                for b in range(S):
                    k = g * S + b

                    @pl.when(k + G < tail0)
                    def _():
                        wait_idx(k + G, (b + G) % S)
                        fire_g((b + G) % S)

                    wait_g(b)
                    fire_s(b)

                    @pl.when(k + I >= S)
                    def _():
                        wait_s((b + I) % S)

                    @pl.when(k + I < tail0)
                    def _():
                        fire_idx(k + I, (b + I) % S)

                return carry

            lax.fori_loop(0, nrounds, round_, 0)
            for t in range(min(S - I, tail0)):
                wait_s((tail0 - 1 - t) % S)

        def tail(k, carry):
            pltpu.sync_copy(eidx_hbm.at[cb + k], eidx.at[0])
            pltpu.async_copy(vals_hbm.at[eidx.at[0, 0]], rows.at[0],
                             gsem).wait()
            pltpu.sync_copy(rows.at[0], acc.at[eidx.at[0, 1]], add=True)
            return carry

        lax.fori_loop(tail0, per_sub, tail, 0)

        @pl.when(s < nextra)
        def _():
            ex = c * per_core + ns * per_sub + s
            pltpu.sync_copy(eidx_hbm.at[ex], exidx)
            pltpu.async_copy(vals_hbm.at[exidx.at[0]], rows.at[0],
                             gsem).wait()
            pltpu.sync_copy(rows.at[0], acc.at[exidx.at[1]], add=True)

        plsc.subcore_barrier()
        if packed:
            pltpu.sync_copy(acc.at[pl.ds(rs, rows_io)],
                            out_hbm.at[pl.ds(rs, rows_io), pl.ds(c * d, d)])
        else:
            pltpu.sync_copy(acc.at[pl.ds(rs, rows_io)],
                            out_hbm.at[c, pl.ds(rs, rows_io)])

    out_shape = ((n, nc * d) if packed else (nc, n, d))
    f = pl.kernel(
        body,
        out_type=jax.ShapeDtypeStruct(out_shape, jnp.float32),
        mesh=_sc_mesh(),
        scratch_types=[
            pltpu.VMEM((S, 2, _CH), jnp.int32),
            pltpu.VMEM((2, _CH), jnp.int32),
            pltpu.VMEM((S, _CH, d), jnp.float32),
            pltpu.VMEM_SHARED((n, d), jnp.float32),
            pltpu.SemaphoreType.DMA,
            pltpu.SemaphoreType.DMA,
            pltpu.SemaphoreType.DMA,
        ],
        compiler_params=pltpu.CompilerParams(use_tc_tiling_on_sc=False),
    )
    return f(vals, eidx)


_BM = 2000


def _dinv_from_deg(deg_ref):
    deg = jnp.sum(deg_ref[...], axis=1) + 1.0
    return lax.rsqrt(deg)[:, None]


def _mm(x, w1, *, n, din, hid):
    def body(x_ref, w_ref, h_ref):
        h_ref[...] = jnp.dot(x_ref[...], w_ref[...],
                             preferred_element_type=jnp.float32)

    return pl.pallas_call(
        body,
        grid=(n // _BM,),
        in_specs=[
            pl.BlockSpec((_BM, din), lambda i: (i, 0)),
            pl.BlockSpec((din, hid), lambda i: (0, 0)),
        ],
        out_specs=pl.BlockSpec((_BM, hid), lambda i: (i, 0)),
        out_shape=jax.ShapeDtypeStruct((n, hid), jnp.float32),
    )(x, w1)


def _scale(h1, degp, *, n, hid):
    def body(h_ref, deg_ref, hs_ref):
        hs_ref[...] = h_ref[...] * _dinv_from_deg(deg_ref)

    return pl.pallas_call(
        body,
        grid=(n // _BM,),
        in_specs=[
            pl.BlockSpec((_BM, hid), lambda i: (i, 0)),
            pl.BlockSpec((_BM, 32), lambda i: (i, 0)),
        ],
        out_specs=pl.BlockSpec((_BM, hid), lambda i: (i, 0)),
        out_shape=jax.ShapeDtypeStruct((n, hid), jnp.float32),
    )(h1, degp)


def _mid(p, hs, eps, degp, w2a, w2b, b1, mean, lsd, *, n, hid, dout):
    def body(p_ref, hs_ref, eps_ref, deg_ref, w2a_ref, w2b_ref, b1_ref,
             mean_ref, lsd_ref, out_ref):
        dinv = _dinv_from_deg(deg_ref)
        agg = p_ref[0] + p_ref[1] - hs_ref[...]
        h = jnp.maximum(dinv * agg + b1_ref[...], 0.0)
        c = jnp.exp(lsd_ref[...]) * eps_ref[...] + mean_ref[...]
        g = (jnp.dot(h - c, w2a_ref[...], preferred_element_type=jnp.float32)
             + jnp.dot(c, w2b_ref[...], preferred_element_type=jnp.float32))
        out_ref[...] = g * dinv

    return pl.pallas_call(
        body,
        grid=(n // _BM,),
        in_specs=[
            pl.BlockSpec((2, _BM, hid), lambda i: (0, i, 0)),
            pl.BlockSpec((_BM, hid), lambda i: (i, 0)),
            pl.BlockSpec((_BM, hid), lambda i: (i, 0)),
            pl.BlockSpec((_BM, 32), lambda i: (i, 0)),
            pl.BlockSpec((hid, dout), lambda i: (0, 0)),
            pl.BlockSpec((hid, dout), lambda i: (0, 0)),
            pl.BlockSpec((1, hid), lambda i: (0, 0)),
            pl.BlockSpec((1, hid), lambda i: (0, 0)),
            pl.BlockSpec((1, hid), lambda i: (0, 0)),
        ],
        out_specs=pl.BlockSpec((_BM, dout), lambda i: (i, 0)),
        out_shape=jax.ShapeDtypeStruct((n, dout), jnp.float32),
    )(p, hs, eps, degp, w2a, w2b, b1, mean, lsd)


def _fin(q, gs, degp, b2, *, n, dout):
    def body(q_ref, gs_ref, deg_ref, b2_ref, out_ref):
        dinv = _dinv_from_deg(deg_ref)
        agg = q_ref[:, :dout] + q_ref[:, dout:] - gs_ref[...]
        out_ref[...] = dinv * agg + b2_ref[...]

    return pl.pallas_call(
        body,
        grid=(n // _BM,),
        in_specs=[
            pl.BlockSpec((_BM, 2 * dout), lambda i: (i, 0)),
            pl.BlockSpec((_BM, dout), lambda i: (i, 0)),
            pl.BlockSpec((_BM, 32), lambda i: (i, 0)),
            pl.BlockSpec((1, dout), lambda i: (0, 0)),
        ],
        out_specs=pl.BlockSpec((_BM, dout), lambda i: (i, 0)),
        out_shape=jax.ShapeDtypeStruct((n, dout), jnp.float32),
    )(q, gs, degp, b2)


def kernel(x, edge_index, W1, b1, mean, log_std_dev, W2, b2, epsilon):
    n, din = x.shape
    hid = W1.shape[1]
    dout = W2.shape[1]
    e = edge_index.shape[1]

    eidx = jnp.swapaxes(edge_index.reshape(2, e // _CH, _CH), 0, 1)
    ones_rows = jnp.full((_CH, 16), 1.0 / 16.0, dtype=jnp.float32)
    zeros16 = jnp.zeros((n, 16), dtype=jnp.float32)

    degp = _deg_partials(eidx, ones_rows, zeros16, n=n, e=e)
    h1 = _mm(x, W1, n=n, din=din, hid=hid)
    hs = _scale(h1, degp, n=n, hid=hid)
    p = _edge_agg(hs, eidx, n=n, d=hid, e=e)
    gs = _mid(p, hs, epsilon, degp, W2[:hid], W2[hid:],
              b1.reshape(1, hid), mean.reshape(1, hid),
              log_std_dev.reshape(1, hid), n=n, hid=hid, dout=dout)
    q = _edge_agg(gs, eidx, n=n, d=dout, e=e)
    return _fin(q, gs, degp, b2.reshape(1, dout), n=n, dout=dout)
